# Initial kernel scaffold; baseline (speedup 1.0000x reference)
import functools

import jax
import jax.numpy as jnp
from jax import lax
from jax.experimental import pallas as pl
from jax.experimental.pallas import tpu as pltpu
from jax.experimental.pallas import tpu_sc as plsc

D = 64
V = 100000
B = 16384
NEG = 5
K = NEG + 1
NC = 2
NS = 16
NW = NC * NS
CH = 128
NCH = B // CH
CPW = NCH // NW
L = 16
NG = CH // L

_mesh = plsc.VectorSubcoreMesh(core_axis_name="c", subcore_axis_name="s")


@functools.partial(
    pl.kernel,
    out_type=jax.ShapeDtypeStruct((NCH, K, CH), jnp.float32),
    mesh=_mesh,
    scratch_types=[
        pltpu.VMEM((K, CH), jnp.int32),
        pltpu.VMEM((K * CH, D), jnp.float32),
        pltpu.VMEM((D, CH), jnp.float32),
        pltpu.VMEM((K, CH), jnp.float32),
        pltpu.SemaphoreType.DMA,
    ],
)
def _sc_score(idx_hbm, h_hbm, table_hbm, out_hbm,
              idx_v, rows_v, h_v, scores_v, sem):
    wid = lax.axis_index("s") * NC + lax.axis_index("c")
    lane = lax.iota(jnp.int32, (L,))
    rid0 = lane * K

    @pl.loop(0, CPW)
    def _chunk(c):
        s = wid * CPW + c
        pltpu.sync_copy(idx_hbm.at[s], idx_v)
        descs = [
            pltpu.async_copy(table_hbm.at[idx_v.at[k]],
                             rows_v.at[pl.ds(k * CH, CH)], sem)
            for k in range(K)
        ]
        pltpu.sync_copy(h_hbm.at[s], h_v)
        for d_ in descs:
            d_.wait()

        @pl.loop(0, NG)
        def _group(g):
            b0 = g * L
            rbase = b0 * K + rid0
            accs = [jnp.zeros((L,), jnp.float32) for _ in range(K)]
            for d in range(D):
                hv = h_v[d, pl.ds(b0, L)]
                dcol = jnp.full((L,), d, jnp.int32)
                for k in range(K):
                    wv = plsc.load_gather(rows_v, [rbase + k, dcol])
                    accs[k] = accs[k] + hv * wv
            for k in range(K):
                score = 1.0 / (1.0 + jnp.exp(-accs[k]))
                scores_v[k, pl.ds(b0, L)] = score

        pltpu.sync_copy(scores_v, out_hbm.at[s])


_NEG_CACHE = None


def _neg_indices():
    global _NEG_CACHE
    if _NEG_CACHE is None:
        _NEG_CACHE = jax.random.randint(
            jax.random.key(123), (B, NEG), 0, V).astype(jnp.int32)
    return _NEG_CACHE


def kernel(h, target_index, embedding_weight):
    neg_idx = _neg_indices()
    idx_cat = jnp.concatenate(
        [target_index.astype(jnp.int32)[:, None], neg_idx], axis=1)
    idx_prep = idx_cat.reshape(B * K).reshape(NCH, K, CH)
    h_prep = h.reshape(NCH, CH, D).transpose(0, 2, 1)
    out = _sc_score(idx_prep, h_prep, embedding_weight)
    o = out.transpose(1, 0, 2).reshape(K, B)
    pos_out = o[0].reshape(B, 1)
    neg_out = o[1:].T
    pos_label = jnp.ones((B, 1), dtype=jnp.float32)
    neg_label = jnp.zeros((B, NEG), dtype=jnp.float32)
    return (pos_out, pos_label, neg_out, neg_label)


# speedup vs baseline: 1.1098x; 1.1098x over previous
import functools

import jax
import jax.numpy as jnp
from jax import lax
from jax.experimental import pallas as pl
from jax.experimental.pallas import tpu as pltpu
from jax.experimental.pallas import tpu_sc as plsc

D = 64
V = 100000
B = 16384
NEG = 5
K = NEG + 1
NC = 2
NS = 16
NW = NC * NS
CH = 128
NCH = B // CH
CPW = NCH // NW
L = 16
NG = CH // L

_mesh = plsc.VectorSubcoreMesh(core_axis_name="c", subcore_axis_name="s")


@functools.partial(
    pl.kernel,
    out_type=jax.ShapeDtypeStruct((NCH, K, CH), jnp.float32),
    mesh=_mesh,
    scratch_types=[
        pltpu.VMEM((K, CH), jnp.int32),
        pltpu.VMEM((K * CH, D), jnp.float32),
        pltpu.VMEM((D, CH), jnp.float32),
        pltpu.VMEM((K, CH), jnp.float32),
        pltpu.SemaphoreType.DMA,
    ],
    compiler_params=pltpu.CompilerParams(
        needs_layout_passes=False, use_tc_tiling_on_sc=False),
)
def _sc_score(idx_hbm, h_hbm, table_hbm, out_hbm,
              idx_v, rows_v, h_v, scores_v, sem):
    wid = lax.axis_index("s") * NC + lax.axis_index("c")
    lane = lax.iota(jnp.int32, L)
    rid0 = lane * K

    @pl.loop(0, CPW)
    def _chunk(c):
        s = wid * CPW + c
        pltpu.sync_copy(idx_hbm.at[s], idx_v)
        descs = [
            pltpu.async_copy(table_hbm.at[idx_v.at[k]],
                             rows_v.at[pl.ds(k * CH, CH)], sem)
            for k in range(K)
        ]
        pltpu.sync_copy(h_hbm.at[s], h_v)
        for d_ in descs:
            d_.wait()

        @pl.loop(0, NG)
        def _group(g):
            b0 = g * L
            rbase = b0 * K + rid0
            accs = [jnp.zeros((L,), jnp.float32) for _ in range(K)]
            for d in range(D):
                hv = h_v[d, pl.ds(b0, L)]
                dcol = jnp.full((L,), d, jnp.int32)
                for k in range(K):
                    wv = plsc.load_gather(rows_v, [rbase + k, dcol])
                    accs[k] = accs[k] + hv * wv
            for k in range(K):
                score = 1.0 / (1.0 + jnp.exp(-accs[k]))
                scores_v[k, pl.ds(b0, L)] = score

        pltpu.sync_copy(scores_v, out_hbm.at[s])


_NEG_CACHE = None


def _neg_indices():
    global _NEG_CACHE
    if _NEG_CACHE is None:
        _NEG_CACHE = jax.random.randint(
            jax.random.key(123), (B, NEG), 0, V).astype(jnp.int32)
    return _NEG_CACHE


def kernel(h, target_index, embedding_weight):
    neg_idx = _neg_indices()
    idx_cat = jnp.concatenate(
        [target_index.astype(jnp.int32)[:, None], neg_idx], axis=1)
    idx_prep = idx_cat.reshape(B * K).reshape(NCH, K, CH)
    h_prep = h.reshape(NCH, CH, D).transpose(0, 2, 1)
    out = _sc_score(idx_prep, h_prep, embedding_weight)
    o = out.transpose(1, 0, 2).reshape(K, B)
    pos_out = o[0].reshape(B, 1)
    neg_out = o[1:].T
    pos_label = jnp.ones((B, 1), dtype=jnp.float32)
    neg_label = jnp.zeros((B, NEG), dtype=jnp.float32)
    return (pos_out, pos_label, neg_out, neg_label)


# speedup vs baseline: 2.1451x; 1.9329x over previous
import functools

import jax
import jax.numpy as jnp
from jax import lax
from jax.experimental import pallas as pl
from jax.experimental.pallas import tpu as pltpu
from jax.experimental.pallas import tpu_sc as plsc

D = 64
V = 100000
B = 16384
NEG = 5
K = NEG + 1
NC = 2
NS = 16
NW = NC * NS
CH = 128
NCH = B // CH
CPW = NCH // NW
L = 16
NG = CH // L

_mesh = plsc.VectorSubcoreMesh(core_axis_name="c", subcore_axis_name="s")


@functools.partial(
    pl.kernel,
    out_type=jax.ShapeDtypeStruct((NCH, K, CH), jnp.float32),
    mesh=_mesh,
    scratch_types=[
        pltpu.VMEM((CPW, K, CH), jnp.int32),
        pltpu.VMEM((K * CH, D), jnp.float32),
        pltpu.VMEM((K * CH, D), jnp.float32),
        pltpu.VMEM((D, CH), jnp.float32),
        pltpu.VMEM((D, CH), jnp.float32),
        pltpu.VMEM((K, CH), jnp.float32),
        pltpu.VMEM((K, CH), jnp.float32),
        pltpu.SemaphoreType.DMA,
        pltpu.SemaphoreType.DMA,
        pltpu.SemaphoreType.DMA,
    ],
    compiler_params=pltpu.CompilerParams(
        needs_layout_passes=False, use_tc_tiling_on_sc=False),
)
def _sc_score(idx_hbm, h_hbm, table_hbm, out_hbm,
              idx_v, rows0, rows1, h0, h1, sc0, sc1,
              sem0, sem1, sem_out):
    wid = lax.axis_index("s") * NC + lax.axis_index("c")
    lane = lax.iota(jnp.int32, L)
    rows_bufs = (rows0, rows1)
    h_bufs = (h0, h1)
    sc_bufs = (sc0, sc1)
    sems = (sem0, sem1)

    pltpu.sync_copy(idx_hbm.at[pl.ds(wid * CPW, CPW)], idx_v)

    def fire(c):
        p = c % 2
        descs = [
            pltpu.async_copy(table_hbm.at[idx_v.at[c, k]],
                             rows_bufs[p].at[pl.ds(k * CH, CH)], sems[p])
            for k in range(K)
        ]
        descs.append(
            pltpu.async_copy(h_hbm.at[wid * CPW + c], h_bufs[p], sems[p]))
        return descs

    pending = {0: fire(0)}
    out_descs = []
    for c in range(CPW):
        p = c % 2
        if c + 1 < CPW:
            pending[c + 1] = fire(c + 1)
        for d_ in pending.pop(c):
            d_.wait()
        rows_v, h_v, scores_v = rows_bufs[p], h_bufs[p], sc_bufs[p]
        if c >= 2:
            out_descs[c - 2].wait()

        @pl.loop(0, NG)
        def _group(g):
            b0 = g * L
            bvec = b0 + lane
            rvecs = [k * CH + bvec for k in range(K)]
            zero = jnp.zeros((L,), jnp.float32)

            @pl.loop(0, D, init_carry=(lane,) + (zero,) * K, unroll=4)
            def _dstep(t, carry):
                dvec, *accs = carry
                hv = plsc.load_gather(h_v, [dvec, bvec])
                new_accs = [
                    accs[k] + hv * plsc.load_gather(rows_v, [rvecs[k], dvec])
                    for k in range(K)
                ]
                dvec = jnp.bitwise_and(dvec + 1, D - 1)
                return (dvec, *new_accs)

            accs = _dstep[1:]
            for k in range(K):
                score = 1.0 / (1.0 + jnp.exp(-accs[k]))
                scores_v[k, pl.ds(b0, L)] = score

        out_descs.append(
            pltpu.async_copy(scores_v, out_hbm.at[wid * CPW + c], sem_out))
    for d_ in out_descs[-2:]:
        d_.wait()


_NEG_CACHE = None


def _neg_indices():
    global _NEG_CACHE
    if _NEG_CACHE is None:
        _NEG_CACHE = jax.random.randint(
            jax.random.key(123), (B, NEG), 0, V).astype(jnp.int32)
    return _NEG_CACHE


def kernel(h, target_index, embedding_weight):
    neg_idx = _neg_indices()
    idx_cat = jnp.concatenate(
        [target_index.astype(jnp.int32)[:, None], neg_idx], axis=1)
    idx_prep = idx_cat.reshape(NCH, CH, K).transpose(0, 2, 1)
    h_prep = h.reshape(NCH, CH, D).transpose(0, 2, 1)
    out = _sc_score(idx_prep, h_prep, embedding_weight)
    o = out.transpose(1, 0, 2).reshape(K, B)
    pos_out = o[0].reshape(B, 1)
    neg_out = o[1:].T
    pos_label = jnp.ones((B, 1), dtype=jnp.float32)
    neg_label = jnp.zeros((B, NEG), dtype=jnp.float32)
    return (pos_out, pos_label, neg_out, neg_label)
